# Initial kernel scaffold; baseline (speedup 1.0000x reference)
import jax
import jax.numpy as jnp
from jax.experimental import pallas as pl


def kernel(g, nfeats, efeats, emb_table, enc_W, enc_b, edge_W, edge_b, proj_W, proj_b, ef_W1, ef_b1, ef_W2, ef_b2, nn_bias, gru_Wih, gru_bih, gru_Whh, gru_bhh, dec_W1, dec_b1, dec_a1, dec_W2, dec_b2, dec_a2, dec_W3, dec_b3, dec_a3, dec_W4, dec_b4):
    raise NotImplementedError("write your pallas kernel here")


# speedup vs baseline: 2.8615x; 2.8615x over previous
import functools

import jax
import jax.numpy as jnp
from jax import lax
from jax.experimental import pallas as pl
from jax.experimental.pallas import tpu as pltpu
from jax.experimental.pallas import tpu_sc as plsc

N = 10000
E = 160000
EMB = 32
NTY = 100
STEPS = 3

NC, NS = 2, 16
NW = NC * NS
IDXW = 128
ROWS_PER_W = 40
EPAD = NW * ROWS_PER_W * IDXW
EDGES_PER_W = ROWS_PER_W * IDXW
GRP = 8
NGRP = ROWS_PER_W // GRP
NPAD = 10016
ZSTR = NPAD // NS
OSTR = N // NS

_f32 = jnp.float32


def _sc_gather_body(h_hbm, src_hbm, hs_hbm, idx_v, rows_v, sem):
    c = lax.axis_index("c")
    s = lax.axis_index("s")
    w = s * NC + c
    row0 = w * ROWS_PER_W
    pltpu.sync_copy(src_hbm.at[pl.ds(row0, ROWS_PER_W)], idx_v)

    def grp(g, carry):
        cps = []
        for b in range(GRP):
            cps.append(pltpu.async_copy(
                h_hbm.at[idx_v.at[g * GRP + b]],
                rows_v.at[pl.ds(b * IDXW, IDXW)], sem))
        for cp in cps:
            cp.wait()
        pltpu.sync_copy(
            rows_v,
            hs_hbm.at[pl.ds(w * EDGES_PER_W + g * (GRP * IDXW), GRP * IDXW)])
        return carry

    lax.fori_loop(0, NGRP, grp, 0)


@functools.cache
def _sc_gather_kernel():
    return pl.kernel(
        _sc_gather_body,
        out_type=jax.ShapeDtypeStruct((EPAD, EMB), _f32),
        mesh=plsc.VectorSubcoreMesh(core_axis_name="c", subcore_axis_name="s"),
        scratch_types=[
            pltpu.VMEM((ROWS_PER_W, IDXW), jnp.int32),
            pltpu.VMEM((GRP * IDXW, EMB), _f32),
            pltpu.SemaphoreType.DMA,
        ],
        compiler_params=pltpu.CompilerParams(use_tc_tiling_on_sc=False),
    )


def _sc_gather(h, src2d):
    return _sc_gather_kernel()(h, src2d)


def _sc_scatter_body(m_hbm, dst_hbm, zeros_hbm, agg_hbm, idx_v, vals_v, zbuf_v,
                     acc_sh):
    c = lax.axis_index("c")
    s = lax.axis_index("s")
    w = s * NC + c
    pltpu.sync_copy(zeros_hbm.at[pl.ds(s * ZSTR, ZSTR)], zbuf_v)
    pltpu.sync_copy(zbuf_v, acc_sh.at[pl.ds(s * ZSTR, ZSTR)])
    plsc.subcore_barrier()

    row0 = w * ROWS_PER_W
    pltpu.sync_copy(dst_hbm.at[pl.ds(row0, ROWS_PER_W)], idx_v)

    def grp(g, carry):
        base = w * EDGES_PER_W + g * (GRP * IDXW)
        pltpu.sync_copy(m_hbm.at[pl.ds(base, GRP * IDXW)], vals_v)
        for b in range(GRP):
            pltpu.sync_copy(vals_v.at[pl.ds(b * IDXW, IDXW)],
                            acc_sh.at[idx_v.at[g * GRP + b]], add=True)
        return carry

    lax.fori_loop(0, NGRP, grp, 0)
    plsc.subcore_barrier()
    pltpu.sync_copy(acc_sh.at[pl.ds(s * OSTR, OSTR)],
                    agg_hbm.at[pl.ds(c * N + s * OSTR, OSTR)])


@functools.cache
def _sc_scatter_kernel():
    return pl.kernel(
        _sc_scatter_body,
        out_type=jax.ShapeDtypeStruct((2 * N, EMB), _f32),
        mesh=plsc.VectorSubcoreMesh(core_axis_name="c", subcore_axis_name="s"),
        scratch_types=[
            pltpu.VMEM((ROWS_PER_W, IDXW), jnp.int32),
            pltpu.VMEM((GRP * IDXW, EMB), _f32),
            pltpu.VMEM((ZSTR, EMB), _f32),
            pltpu.VMEM_SHARED((NPAD, EMB), _f32),
        ],
        compiler_params=pltpu.CompilerParams(use_tc_tiling_on_sc=False),
    )


def _sc_scatter(m, dst2d, zeros_acc):
    return _sc_scatter_kernel()(m, dst2d, zeros_acc)


def _dot(a, b):
    return jnp.dot(a, b, preferred_element_type=_f32)


def _encode_body(nt_ref, emb_ref, encw_ref, encb_ref, projw_ref, projb_ref,
                 h_ref):
    nt = nt_ref[...]
    ids = lax.broadcasted_iota(jnp.int32, (N, 128), 1)
    oh = jnp.where(nt == ids, 1.0, 0.0).astype(_f32)
    nh = jax.nn.relu(_dot(oh, emb_ref[...]))
    nh = jax.nn.relu(_dot(nh, encw_ref[...]) + encb_ref[...])
    h_ref[...] = jax.nn.relu(_dot(nh, projw_ref[...]) + projb_ref[...])


def _encode(nt, emb_pad, enc_W, enc_b, proj_W, proj_b):
    return pl.pallas_call(
        _encode_body,
        out_shape=jax.ShapeDtypeStruct((N, EMB), _f32),
    )(nt, emb_pad, enc_W, enc_b, proj_W, proj_b)


EB_R = 8192


def _r2_body(ef_ref, ew_ref, eb_ref, w1_ref, b1_ref, r2_ref):
    eh = _dot(ef_ref[...], ew_ref[...]) + eb_ref[...]
    r2_ref[...] = jax.nn.relu(_dot(eh, w1_ref[...]) + b1_ref[...])


def _r2(efp, edge_W, edge_b, ef_W1, ef_b1):
    grid = EPAD // EB_R
    return pl.pallas_call(
        _r2_body,
        grid=(grid,),
        in_specs=[
            pl.BlockSpec((EB_R, 16), lambda i: (i, 0)),
            pl.BlockSpec((16, EMB), lambda i: (0, 0)),
            pl.BlockSpec((1, EMB), lambda i: (0, 0)),
            pl.BlockSpec((EMB, EMB), lambda i: (0, 0)),
            pl.BlockSpec((1, EMB), lambda i: (0, 0)),
        ],
        out_specs=pl.BlockSpec((EB_R, EMB), lambda i: (i, 0)),
        out_shape=jax.ShapeDtypeStruct((EPAD, EMB), _f32),
    )(efp, edge_W, edge_b, ef_W1, ef_b1)


EB_M = 2048


def _msg_body(hs_ref, r2_ref, vcat_ref, s_ref, b2_ref, m_ref):
    hs = hs_ref[...]
    r2 = r2_ref[...]
    g = _dot(hs, vcat_ref[...])
    z = g * jnp.concatenate([r2] * EMB, axis=1)
    m_ref[...] = _dot(z, s_ref[...]) + _dot(hs, b2_ref[...])


def _msg(hs, r2, vcat, smat, b2):
    grid = EPAD // EB_M
    return pl.pallas_call(
        _msg_body,
        grid=(grid,),
        in_specs=[
            pl.BlockSpec((EB_M, EMB), lambda i: (i, 0)),
            pl.BlockSpec((EB_M, EMB), lambda i: (i, 0)),
            pl.BlockSpec((EMB, EMB * EMB), lambda i: (0, 0)),
            pl.BlockSpec((EMB * EMB, EMB), lambda i: (0, 0)),
            pl.BlockSpec((EMB, EMB), lambda i: (0, 0)),
        ],
        out_specs=pl.BlockSpec((EB_M, EMB), lambda i: (i, 0)),
        out_shape=jax.ShapeDtypeStruct((EPAD, EMB), _f32),
    )(hs, r2, vcat, smat, b2)


def _gru_body(ap_ref, hid_ref, nnb_ref, wih_ref, bih_ref, whh_ref, bhh_ref,
              out_ref):
    agg = ap_ref[0:N, :] + ap_ref[N:2 * N, :] + nnb_ref[...]
    h = jax.nn.relu(agg)
    hidden = hid_ref[...]
    gi = _dot(h, wih_ref[...]) + bih_ref[...]
    gh = _dot(hidden, whh_ref[...]) + bhh_ref[...]
    r = jax.nn.sigmoid(gi[:, 0:EMB] + gh[:, 0:EMB])
    z = jax.nn.sigmoid(gi[:, EMB:2 * EMB] + gh[:, EMB:2 * EMB])
    ng = jnp.tanh(gi[:, 2 * EMB:3 * EMB] + r * gh[:, 2 * EMB:3 * EMB])
    out_ref[...] = (1.0 - z) * ng + z * hidden


def _gru(aggp, hidden, nn_bias, wih_t, bih, whh_t, bhh):
    return pl.pallas_call(
        _gru_body,
        out_shape=jax.ShapeDtypeStruct((N, EMB), _f32),
    )(aggp, hidden, nn_bias, wih_t, bih, whh_t, bhh)


def _dec_body(h_ref, w1_ref, b1_ref, a1_ref, w2_ref, b2_ref, a2_ref, w3_ref,
              b3_ref, a3_ref, w4_ref, b4_ref, out_ref):
    def prelu(x, a_ref):
        return jnp.where(x >= 0, x, x * a_ref[...])

    h = h_ref[...]
    h = prelu(_dot(h, w1_ref[...]) + b1_ref[...], a1_ref)
    h = prelu(_dot(h, w2_ref[...]) + b2_ref[...], a2_ref)
    h = prelu(_dot(h, w3_ref[...]) + b3_ref[...], a3_ref)
    out_ref[...] = _dot(h, w4_ref[...]) + b4_ref[...]


def _decode(h, dec_W1, dec_b1, dec_a1, dec_W2, dec_b2, dec_a2, dec_W3, dec_b3,
            dec_a3, dec_W4, dec_b4):
    return pl.pallas_call(
        _dec_body,
        out_shape=jax.ShapeDtypeStruct((N, EMB), _f32),
    )(h, dec_W1, dec_b1, dec_a1, dec_W2, dec_b2, dec_a2, dec_W3, dec_b3,
      dec_a3, dec_W4, dec_b4)


def kernel(g, nfeats, efeats, emb_table, enc_W, enc_b, edge_W, edge_b, proj_W,
           proj_b, ef_W1, ef_b1, ef_W2, ef_b2, nn_bias, gru_Wih, gru_bih,
           gru_Whh, gru_bhh, dec_W1, dec_b1, dec_a1, dec_W2, dec_b2, dec_a2,
           dec_W3, dec_b3, dec_a3, dec_W4, dec_b4):
    src = g[0]
    dst = g[1]
    npad = EPAD - E
    src2d = jnp.concatenate([src, jnp.zeros((npad,), jnp.int32)]).reshape(
        NW * ROWS_PER_W, IDXW)
    dst2d = jnp.concatenate([dst, jnp.full((npad,), N, jnp.int32)]).reshape(
        NW * ROWS_PER_W, IDXW)
    efp = jnp.pad(efeats, ((0, npad), (0, 0)))
    emb_pad = jnp.pad(emb_table, ((0, 128 - NTY), (0, 0)))
    zeros_acc = jnp.zeros((NPAD, EMB), _f32)

    vcat = ef_W2.reshape(EMB, EMB, EMB).transpose(1, 2, 0).reshape(
        EMB, EMB * EMB)
    smat = jnp.repeat(jnp.eye(EMB, dtype=_f32), EMB, axis=0)
    b2m = ef_b2.reshape(EMB, EMB)
    wih_t = gru_Wih.T
    whh_t = gru_Whh.T

    r1 = lambda v: v.reshape(1, -1)
    r11 = lambda v: v.reshape(1, 1)

    h = _encode(nfeats, emb_pad, enc_W, r1(enc_b), proj_W, r1(proj_b))
    r2 = _r2(efp, edge_W, r1(edge_b), ef_W1, r1(ef_b1))
    hidden = h
    for _ in range(STEPS):
        hs = _sc_gather(h, src2d)
        m = _msg(hs, r2, vcat, smat, b2m)
        aggp = _sc_scatter(m, dst2d, zeros_acc)
        hidden = _gru(aggp, hidden, r1(nn_bias), wih_t, r1(gru_bih), whh_t,
                      r1(gru_bhh))
        h = hidden
    return _decode(hidden, dec_W1, r1(dec_b1), r11(dec_a1), dec_W2, r1(dec_b2),
                   r11(dec_a2), dec_W3, r1(dec_b3), r11(dec_a3), dec_W4,
                   r1(dec_b4))


# speedup vs baseline: 2.8992x; 1.0132x over previous
import functools

import jax
import jax.numpy as jnp
from jax import lax
from jax.experimental import pallas as pl
from jax.experimental.pallas import tpu as pltpu
from jax.experimental.pallas import tpu_sc as plsc

N = 10000
E = 160000
EMB = 32
NTY = 100
STEPS = 3

NC, NS = 2, 16
NW = NC * NS
IDXW = 128
ROWS_PER_W = 40
EPAD = NW * ROWS_PER_W * IDXW
EDGES_PER_W = ROWS_PER_W * IDXW
GRP = 8
NGRP = ROWS_PER_W // GRP
NPAD = 10016
ZSTR = NPAD // NS
OSTR = N // NS

_f32 = jnp.float32


CHE = 1024
NCH = EDGES_PER_W // CHE
NB = 3


def _sc_gather_body(h_hbm, src_hbm, hs_hbm, idx_v, rows_v, g0, g1, g2, w0, w1,
                    w2):
    gsems = [g0, g1, g2]
    wsems = [w0, w1, w2]
    c = lax.axis_index("c")
    s = lax.axis_index("s")
    w = s * NC + c
    pltpu.sync_copy(src_hbm.at[pl.ds(w * NCH, NCH)], idx_v)

    gd = [None] * NCH
    wd = [None] * NCH

    def fire(g):
        b = g % NB
        gd[g] = pltpu.async_copy(h_hbm.at[idx_v.at[g]], rows_v.at[b],
                                 gsems[b])

    for g in range(NB):
        fire(g)
    for g in range(NCH):
        b = g % NB
        gd[g].wait()
        wd[g] = pltpu.async_copy(
            rows_v.at[b], hs_hbm.at[pl.ds((w * NCH + g) * CHE, CHE)],
            wsems[b])
        if g + NB < NCH:
            wd[g].wait()
            fire(g + NB)
    for g in range(max(0, NCH - NB), NCH):
        wd[g].wait()


@functools.cache
def _sc_gather_kernel():
    return pl.kernel(
        _sc_gather_body,
        out_type=jax.ShapeDtypeStruct((EPAD, EMB), _f32),
        mesh=plsc.VectorSubcoreMesh(core_axis_name="c", subcore_axis_name="s"),
        scratch_types=[
            pltpu.VMEM((NCH, CHE), jnp.int32),
            pltpu.VMEM((NB, CHE, EMB), _f32),
            pltpu.SemaphoreType.DMA,
            pltpu.SemaphoreType.DMA,
            pltpu.SemaphoreType.DMA,
            pltpu.SemaphoreType.DMA,
            pltpu.SemaphoreType.DMA,
            pltpu.SemaphoreType.DMA,
        ],
        compiler_params=pltpu.CompilerParams(use_tc_tiling_on_sc=False),
    )


def _sc_gather(h, src2d):
    return _sc_gather_kernel()(h, src2d)


NBS = 2


def _sc_scatter_body(m_hbm, dst_hbm, zeros_hbm, agg_hbm, idx_v, vals_v,
                     acc_sh, l0, l1, a0, a1):
    lsems = [l0, l1]
    asems = [a0, a1]
    c = lax.axis_index("c")
    s = lax.axis_index("s")
    w = s * NC + c
    pltpu.sync_copy(zeros_hbm.at[pl.ds(s * ZSTR, ZSTR)],
                    vals_v.at[0].at[pl.ds(0, ZSTR)])
    pltpu.sync_copy(vals_v.at[0].at[pl.ds(0, ZSTR)],
                    acc_sh.at[pl.ds(s * ZSTR, ZSTR)])
    pltpu.sync_copy(dst_hbm.at[pl.ds(w * NCH, NCH)], idx_v)

    ld = [None] * NCH
    ad = [None] * NCH

    def load(g):
        b = g % NBS
        ld[g] = pltpu.async_copy(
            m_hbm.at[pl.ds((w * NCH + g) * CHE, CHE)], vals_v.at[b],
            lsems[b])

    for g in range(NBS):
        load(g)
    plsc.subcore_barrier()
    for g in range(NCH):
        b = g % NBS
        ld[g].wait()
        ad[g] = pltpu.async_copy(vals_v.at[b], acc_sh.at[idx_v.at[g]],
                                 asems[b], add=True)
        if g + NBS < NCH:
            ad[g].wait()
            load(g + NBS)
    for g in range(max(0, NCH - NBS), NCH):
        ad[g].wait()
    plsc.subcore_barrier()
    pltpu.sync_copy(acc_sh.at[pl.ds(s * OSTR, OSTR)],
                    agg_hbm.at[pl.ds(c * N + s * OSTR, OSTR)])


@functools.cache
def _sc_scatter_kernel():
    return pl.kernel(
        _sc_scatter_body,
        out_type=jax.ShapeDtypeStruct((2 * N, EMB), _f32),
        mesh=plsc.VectorSubcoreMesh(core_axis_name="c", subcore_axis_name="s"),
        scratch_types=[
            pltpu.VMEM((NCH, CHE), jnp.int32),
            pltpu.VMEM((NBS, CHE, EMB), _f32),
            pltpu.VMEM_SHARED((NPAD, EMB), _f32),
            pltpu.SemaphoreType.DMA,
            pltpu.SemaphoreType.DMA,
            pltpu.SemaphoreType.DMA,
            pltpu.SemaphoreType.DMA,
        ],
        compiler_params=pltpu.CompilerParams(use_tc_tiling_on_sc=False),
    )


def _sc_scatter(m, dst2d, zeros_acc):
    return _sc_scatter_kernel()(m, dst2d, zeros_acc)


def _dot(a, b):
    return jnp.dot(a, b, preferred_element_type=_f32)


def _encode_body(nt_ref, emb_ref, encw_ref, encb_ref, projw_ref, projb_ref,
                 h_ref):
    nt = nt_ref[...]
    ids = lax.broadcasted_iota(jnp.int32, (N, 128), 1)
    oh = jnp.where(nt == ids, 1.0, 0.0).astype(_f32)
    nh = jax.nn.relu(_dot(oh, emb_ref[...]))
    nh = jax.nn.relu(_dot(nh, encw_ref[...]) + encb_ref[...])
    h_ref[...] = jax.nn.relu(_dot(nh, projw_ref[...]) + projb_ref[...])


def _encode(nt, emb_pad, enc_W, enc_b, proj_W, proj_b):
    return pl.pallas_call(
        _encode_body,
        out_shape=jax.ShapeDtypeStruct((N, EMB), _f32),
    )(nt, emb_pad, enc_W, enc_b, proj_W, proj_b)


EB_R = 8192


def _r2_body(ef_ref, ew_ref, eb_ref, w1_ref, b1_ref, r2_ref):
    eh = _dot(ef_ref[...], ew_ref[...]) + eb_ref[...]
    r2_ref[...] = jax.nn.relu(_dot(eh, w1_ref[...]) + b1_ref[...])


def _r2(efp, edge_W, edge_b, ef_W1, ef_b1):
    grid = EPAD // EB_R
    return pl.pallas_call(
        _r2_body,
        grid=(grid,),
        in_specs=[
            pl.BlockSpec((EB_R, 16), lambda i: (i, 0)),
            pl.BlockSpec((16, EMB), lambda i: (0, 0)),
            pl.BlockSpec((1, EMB), lambda i: (0, 0)),
            pl.BlockSpec((EMB, EMB), lambda i: (0, 0)),
            pl.BlockSpec((1, EMB), lambda i: (0, 0)),
        ],
        out_specs=pl.BlockSpec((EB_R, EMB), lambda i: (i, 0)),
        out_shape=jax.ShapeDtypeStruct((EPAD, EMB), _f32),
    )(efp, edge_W, edge_b, ef_W1, ef_b1)


EB_M = 2048


def _msg_body(hs_ref, r2_ref, vcat_ref, s_ref, b2_ref, m_ref):
    hs = hs_ref[...].astype(jnp.bfloat16)
    r2 = r2_ref[...]
    g = _dot(hs, vcat_ref[...])
    z = g * jnp.concatenate([r2] * EMB, axis=1)
    zb = z.astype(jnp.bfloat16)
    m_ref[...] = _dot(zb, s_ref[...]) + _dot(hs, b2_ref[...])


def _msg(hs, r2, vcat, smat, b2):
    grid = EPAD // EB_M
    return pl.pallas_call(
        _msg_body,
        grid=(grid,),
        in_specs=[
            pl.BlockSpec((EB_M, EMB), lambda i: (i, 0)),
            pl.BlockSpec((EB_M, EMB), lambda i: (i, 0)),
            pl.BlockSpec((EMB, EMB * EMB), lambda i: (0, 0)),
            pl.BlockSpec((EMB * EMB, EMB), lambda i: (0, 0)),
            pl.BlockSpec((EMB, EMB), lambda i: (0, 0)),
        ],
        out_specs=pl.BlockSpec((EB_M, EMB), lambda i: (i, 0)),
        out_shape=jax.ShapeDtypeStruct((EPAD, EMB), _f32),
    )(hs, r2, vcat, smat, b2)


def _gru_body(ap_ref, hid_ref, nnb_ref, wih_ref, bih_ref, whh_ref, bhh_ref,
              out_ref):
    agg = ap_ref[0:N, :] + ap_ref[N:2 * N, :] + nnb_ref[...]
    h = jax.nn.relu(agg)
    hidden = hid_ref[...]
    gi = _dot(h, wih_ref[...]) + bih_ref[...]
    gh = _dot(hidden, whh_ref[...]) + bhh_ref[...]
    r = jax.nn.sigmoid(gi[:, 0:EMB] + gh[:, 0:EMB])
    z = jax.nn.sigmoid(gi[:, EMB:2 * EMB] + gh[:, EMB:2 * EMB])
    ng = jnp.tanh(gi[:, 2 * EMB:3 * EMB] + r * gh[:, 2 * EMB:3 * EMB])
    out_ref[...] = (1.0 - z) * ng + z * hidden


def _gru(aggp, hidden, nn_bias, wih_t, bih, whh_t, bhh):
    return pl.pallas_call(
        _gru_body,
        out_shape=jax.ShapeDtypeStruct((N, EMB), _f32),
    )(aggp, hidden, nn_bias, wih_t, bih, whh_t, bhh)


def _dec_body(h_ref, w1_ref, b1_ref, a1_ref, w2_ref, b2_ref, a2_ref, w3_ref,
              b3_ref, a3_ref, w4_ref, b4_ref, out_ref):
    def prelu(x, a_ref):
        return jnp.where(x >= 0, x, x * a_ref[...])

    h = h_ref[...]
    h = prelu(_dot(h, w1_ref[...]) + b1_ref[...], a1_ref)
    h = prelu(_dot(h, w2_ref[...]) + b2_ref[...], a2_ref)
    h = prelu(_dot(h, w3_ref[...]) + b3_ref[...], a3_ref)
    out_ref[...] = _dot(h, w4_ref[...]) + b4_ref[...]


def _decode(h, dec_W1, dec_b1, dec_a1, dec_W2, dec_b2, dec_a2, dec_W3, dec_b3,
            dec_a3, dec_W4, dec_b4):
    return pl.pallas_call(
        _dec_body,
        out_shape=jax.ShapeDtypeStruct((N, EMB), _f32),
    )(h, dec_W1, dec_b1, dec_a1, dec_W2, dec_b2, dec_a2, dec_W3, dec_b3,
      dec_a3, dec_W4, dec_b4)


def kernel(g, nfeats, efeats, emb_table, enc_W, enc_b, edge_W, edge_b, proj_W,
           proj_b, ef_W1, ef_b1, ef_W2, ef_b2, nn_bias, gru_Wih, gru_bih,
           gru_Whh, gru_bhh, dec_W1, dec_b1, dec_a1, dec_W2, dec_b2, dec_a2,
           dec_W3, dec_b3, dec_a3, dec_W4, dec_b4):
    src = g[0]
    dst = g[1]
    npad = EPAD - E
    src2d = jnp.concatenate([src, jnp.zeros((npad,), jnp.int32)]).reshape(
        NW * NCH, CHE)
    dst2d = jnp.concatenate([dst, jnp.full((npad,), N, jnp.int32)]).reshape(
        NW * NCH, CHE)
    efp = jnp.pad(efeats, ((0, npad), (0, 0)))
    emb_pad = jnp.pad(emb_table, ((0, 128 - NTY), (0, 0)))
    zeros_acc = jnp.zeros((NPAD, EMB), _f32)

    vcat = ef_W2.reshape(EMB, EMB, EMB).transpose(1, 2, 0).reshape(
        EMB, EMB * EMB).astype(jnp.bfloat16)
    smat = jnp.repeat(jnp.eye(EMB, dtype=jnp.bfloat16), EMB, axis=0)
    b2m = ef_b2.reshape(EMB, EMB).astype(jnp.bfloat16)
    wih_t = gru_Wih.T
    whh_t = gru_Whh.T

    r1 = lambda v: v.reshape(1, -1)
    r11 = lambda v: v.reshape(1, 1)

    h = _encode(nfeats, emb_pad, enc_W, r1(enc_b), proj_W, r1(proj_b))
    r2 = _r2(efp, edge_W, r1(edge_b), ef_W1, r1(ef_b1))
    hidden = h
    for _ in range(STEPS):
        hs = _sc_gather(h, src2d)
        m = _msg(hs, r2, vcat, smat, b2m)
        aggp = _sc_scatter(m, dst2d, zeros_acc)
        hidden = _gru(aggp, hidden, r1(nn_bias), wih_t, r1(gru_bih), whh_t,
                      r1(gru_bhh))
        h = hidden
    return _decode(hidden, dec_W1, r1(dec_b1), r11(dec_a1), dec_W2, r1(dec_b2),
                   r11(dec_a2), dec_W3, r1(dec_b3), r11(dec_a3), dec_W4,
                   r1(dec_b4))


# speedup vs baseline: 4.5698x; 1.5762x over previous
import functools

import jax
import jax.numpy as jnp
from jax import lax
from jax.experimental import pallas as pl
from jax.experimental.pallas import tpu as pltpu
from jax.experimental.pallas import tpu_sc as plsc

N = 10000
E = 160000
EMB = 32
NTY = 100
STEPS = 3

NC, NS = 2, 16
NW = NC * NS
EDGES_PER_W = E // NW
E4 = E // 4
NPAD = 10016
ZSTR = NPAD // NS
OSTR = N // NS

_f32 = jnp.float32


CHE = 1000
NCH = EDGES_PER_W // CHE
NB = 3


def _sc_gather_body(h_hbm, src_hbm, hs_hbm, idx_v, rows_v, g0, g1, g2, w0, w1,
                    w2):
    gsems = [g0, g1, g2]
    wsems = [w0, w1, w2]
    c = lax.axis_index("c")
    s = lax.axis_index("s")
    w = s * NC + c
    pltpu.sync_copy(src_hbm.at[pl.ds(w * NCH, NCH)], idx_v)

    gd = [None] * NCH
    wd = [None] * NCH

    def fire(g):
        b = g % NB
        gd[g] = pltpu.async_copy(h_hbm.at[idx_v.at[g]], rows_v.at[b],
                                 gsems[b])

    for g in range(NB):
        fire(g)
    for g in range(NCH):
        b = g % NB
        gd[g].wait()
        wd[g] = pltpu.async_copy(
            rows_v.at[b], hs_hbm.at[pl.ds((w * NCH + g) * CHE, CHE)],
            wsems[b])
        if g + NB < NCH:
            wd[g].wait()
            fire(g + NB)
    for g in range(max(0, NCH - NB), NCH):
        wd[g].wait()


@functools.cache
def _sc_gather_kernel():
    return pl.kernel(
        _sc_gather_body,
        out_type=jax.ShapeDtypeStruct((E, EMB), _f32),
        mesh=plsc.VectorSubcoreMesh(core_axis_name="c", subcore_axis_name="s"),
        scratch_types=[
            pltpu.VMEM((NCH, CHE), jnp.int32),
            pltpu.VMEM((NB, CHE, EMB), _f32),
            pltpu.SemaphoreType.DMA,
            pltpu.SemaphoreType.DMA,
            pltpu.SemaphoreType.DMA,
            pltpu.SemaphoreType.DMA,
            pltpu.SemaphoreType.DMA,
            pltpu.SemaphoreType.DMA,
        ],
        compiler_params=pltpu.CompilerParams(use_tc_tiling_on_sc=False),
    )


def _sc_gather(h, src2d):
    return _sc_gather_kernel()(h, src2d)


NBS = 2


def _sc_scatter_body(m_hbm, dst_hbm, zeros_hbm, agg_hbm, idx_v, vals_v,
                     acc_sh, l0, l1, a0, a1):
    lsems = [l0, l1]
    asems = [a0, a1]
    c = lax.axis_index("c")
    s = lax.axis_index("s")
    w = s * NC + c
    pltpu.sync_copy(zeros_hbm.at[pl.ds(s * ZSTR, ZSTR)],
                    vals_v.at[0].at[pl.ds(0, ZSTR)])
    pltpu.sync_copy(vals_v.at[0].at[pl.ds(0, ZSTR)],
                    acc_sh.at[pl.ds(s * ZSTR, ZSTR)])
    pltpu.sync_copy(dst_hbm.at[pl.ds(w * NCH, NCH)], idx_v)

    ld = [None] * NCH
    ad = [None] * NCH

    def load(g):
        b = g % NBS
        ld[g] = pltpu.async_copy(
            m_hbm.at[pl.ds((w * NCH + g) * CHE, CHE)], vals_v.at[b],
            lsems[b])

    for g in range(NBS):
        load(g)
    plsc.subcore_barrier()
    for g in range(NCH):
        b = g % NBS
        ld[g].wait()
        ad[g] = pltpu.async_copy(vals_v.at[b], acc_sh.at[idx_v.at[g]],
                                 asems[b], add=True)
        if g + NBS < NCH:
            ad[g].wait()
            load(g + NBS)
    for g in range(max(0, NCH - NBS), NCH):
        ad[g].wait()
    plsc.subcore_barrier()
    pltpu.sync_copy(acc_sh.at[pl.ds(s * OSTR, OSTR)],
                    agg_hbm.at[pl.ds(c * N + s * OSTR, OSTR)])


@functools.cache
def _sc_scatter_kernel():
    return pl.kernel(
        _sc_scatter_body,
        out_type=jax.ShapeDtypeStruct((2 * N, EMB), _f32),
        mesh=plsc.VectorSubcoreMesh(core_axis_name="c", subcore_axis_name="s"),
        scratch_types=[
            pltpu.VMEM((NCH, CHE), jnp.int32),
            pltpu.VMEM((NBS, CHE, EMB), _f32),
            pltpu.VMEM_SHARED((NPAD, EMB), _f32),
            pltpu.SemaphoreType.DMA,
            pltpu.SemaphoreType.DMA,
            pltpu.SemaphoreType.DMA,
            pltpu.SemaphoreType.DMA,
        ],
        compiler_params=pltpu.CompilerParams(use_tc_tiling_on_sc=False),
    )


def _sc_scatter(m, dst2d, zeros_acc):
    return _sc_scatter_kernel()(m, dst2d, zeros_acc)


def _dot(a, b):
    return jnp.dot(a, b, preferred_element_type=_f32)


def _encode_body(nt_ref, emb_ref, encw_ref, encb_ref, projw_ref, projb_ref,
                 h_ref):
    nt = nt_ref[...]
    ids = lax.broadcasted_iota(jnp.int32, (N, 128), 1)
    oh = jnp.where(nt == ids, 1.0, 0.0).astype(_f32)
    nh = jax.nn.relu(_dot(oh, emb_ref[...]))
    nh = jax.nn.relu(_dot(nh, encw_ref[...]) + encb_ref[...])
    h_ref[...] = jax.nn.relu(_dot(nh, projw_ref[...]) + projb_ref[...])


def _encode(nt, emb_pad, enc_W, enc_b, proj_W, proj_b):
    return pl.pallas_call(
        _encode_body,
        out_shape=jax.ShapeDtypeStruct((N, EMB), _f32),
    )(nt, emb_pad, enc_W, enc_b, proj_W, proj_b)


EB_R = 2000


def _r2_body(ef_ref, ew_ref, eb_ref, w1_ref, b1_ref, r2_ref):
    eh = _dot(ef_ref[...], ew_ref[...]) + eb_ref[...]
    r2_ref[...] = jax.nn.relu(_dot(eh, w1_ref[...]) + b1_ref[...])


def _r2(efp4, edge_W4, edge_b4, w1p, b1p):
    grid = E4 // EB_R
    return pl.pallas_call(
        _r2_body,
        grid=(grid,),
        in_specs=[
            pl.BlockSpec((EB_R, 64), lambda i: (i, 0)),
            pl.BlockSpec((64, 128), lambda i: (0, 0)),
            pl.BlockSpec((1, 128), lambda i: (0, 0)),
            pl.BlockSpec((128, 128), lambda i: (0, 0)),
            pl.BlockSpec((1, 128), lambda i: (0, 0)),
        ],
        out_specs=pl.BlockSpec((EB_R, 128), lambda i: (i, 0)),
        out_shape=jax.ShapeDtypeStruct((E4, 128), _f32),
    )(efp4, edge_W4, edge_b4, w1p, b1p)


EB_M = 800


def _msg_body(hs_ref, r2_ref, vcat_ref, s_ref, b2_ref, m_ref):
    hs = hs_ref[...].astype(jnp.bfloat16)
    r2 = r2_ref[...]
    g = _dot(hs, vcat_ref[...])
    z = g * jnp.concatenate([r2] * EMB, axis=1)
    zb = z.astype(jnp.bfloat16)
    m_ref[...] = _dot(zb, s_ref[...]) + _dot(hs, b2_ref[...])


def _msg(hs4, r2p, vcat, smat, b2):
    grid = E4 // EB_M
    return pl.pallas_call(
        _msg_body,
        grid=(grid,),
        in_specs=[
            pl.BlockSpec((EB_M, 128), lambda i: (i, 0)),
            pl.BlockSpec((EB_M, 128), lambda i: (i, 0)),
            pl.BlockSpec((128, 4 * EMB * EMB), lambda i: (0, 0)),
            pl.BlockSpec((4 * EMB * EMB, 128), lambda i: (0, 0)),
            pl.BlockSpec((128, 128), lambda i: (0, 0)),
        ],
        out_specs=pl.BlockSpec((EB_M, 128), lambda i: (i, 0)),
        out_shape=jax.ShapeDtypeStruct((E4, 128), _f32),
    )(hs4, r2p, vcat, smat, b2)


def _gru_body(ap_ref, hid_ref, nnb_ref, wih_ref, bih_ref, whh_ref, bhh_ref,
              out_ref):
    agg = ap_ref[0:N, :] + ap_ref[N:2 * N, :] + nnb_ref[...]
    h = jax.nn.relu(agg)
    hidden = hid_ref[...]
    gi = _dot(h, wih_ref[...]) + bih_ref[...]
    gh = _dot(hidden, whh_ref[...]) + bhh_ref[...]
    r = jax.nn.sigmoid(gi[:, 0:EMB] + gh[:, 0:EMB])
    z = jax.nn.sigmoid(gi[:, EMB:2 * EMB] + gh[:, EMB:2 * EMB])
    ng = jnp.tanh(gi[:, 2 * EMB:3 * EMB] + r * gh[:, 2 * EMB:3 * EMB])
    out_ref[...] = (1.0 - z) * ng + z * hidden


def _gru(aggp, hidden, nn_bias, wih_t, bih, whh_t, bhh):
    return pl.pallas_call(
        _gru_body,
        out_shape=jax.ShapeDtypeStruct((N, EMB), _f32),
    )(aggp, hidden, nn_bias, wih_t, bih, whh_t, bhh)


def _dec_body(h_ref, w1_ref, b1_ref, a1_ref, w2_ref, b2_ref, a2_ref, w3_ref,
              b3_ref, a3_ref, w4_ref, b4_ref, out_ref):
    def prelu(x, a_ref):
        return jnp.where(x >= 0, x, x * a_ref[...])

    h = h_ref[...]
    h = prelu(_dot(h, w1_ref[...]) + b1_ref[...], a1_ref)
    h = prelu(_dot(h, w2_ref[...]) + b2_ref[...], a2_ref)
    h = prelu(_dot(h, w3_ref[...]) + b3_ref[...], a3_ref)
    out_ref[...] = _dot(h, w4_ref[...]) + b4_ref[...]


def _decode(h, dec_W1, dec_b1, dec_a1, dec_W2, dec_b2, dec_a2, dec_W3, dec_b3,
            dec_a3, dec_W4, dec_b4):
    return pl.pallas_call(
        _dec_body,
        out_shape=jax.ShapeDtypeStruct((N, EMB), _f32),
    )(h, dec_W1, dec_b1, dec_a1, dec_W2, dec_b2, dec_a2, dec_W3, dec_b3,
      dec_a3, dec_W4, dec_b4)


def kernel(g, nfeats, efeats, emb_table, enc_W, enc_b, edge_W, edge_b, proj_W,
           proj_b, ef_W1, ef_b1, ef_W2, ef_b2, nn_bias, gru_Wih, gru_bih,
           gru_Whh, gru_bhh, dec_W1, dec_b1, dec_a1, dec_W2, dec_b2, dec_a2,
           dec_W3, dec_b3, dec_a3, dec_W4, dec_b4):
    src2d = g[0].reshape(NW * NCH, CHE)
    dst2d = g[1].reshape(NW * NCH, CHE)
    efp4 = efeats.reshape(E4, 64)
    emb_pad = jnp.pad(emb_table, ((0, 128 - NTY), (0, 0)))
    zeros_acc = jnp.zeros((NPAD, EMB), _f32)

    bf = jnp.bfloat16
    i4 = jnp.eye(4, dtype=_f32)
    w2r3 = ef_W2.reshape(EMB, EMB, EMB)
    edge_W4 = jnp.einsum('fc,qQ->qfQc', edge_W, i4).reshape(64, 128)
    edge_b4 = jnp.tile(edge_b, 4).reshape(1, 128)
    w1p = jnp.einsum('ck,qQ->qckQ', ef_W1, i4).reshape(128, 128)
    b1p = jnp.repeat(ef_b1, 4).reshape(1, 128)
    vcat = jnp.einsum('kio,qQ->qiokQ', w2r3, i4).reshape(128, 4096).astype(bf)
    smat = jnp.einsum('oO,qQ,k->okqQO', jnp.eye(EMB, dtype=_f32), i4,
                      jnp.ones((EMB,), _f32)).reshape(4096, 128).astype(bf)
    b2m = jnp.einsum('io,qQ->qiQo', ef_b2.reshape(EMB, EMB),
                     i4).reshape(128, 128).astype(bf)
    wih_t = gru_Wih.T
    whh_t = gru_Whh.T

    r1 = lambda v: v.reshape(1, -1)
    r11 = lambda v: v.reshape(1, 1)

    h = _encode(nfeats, emb_pad, enc_W, r1(enc_b), proj_W, r1(proj_b))
    r2p = _r2(efp4, edge_W4, edge_b4, w1p, b1p)
    hidden = h
    for _ in range(STEPS):
        hs = _sc_gather(h, src2d)
        m4 = _msg(hs.reshape(E4, 128), r2p, vcat, smat, b2m)
        aggp = _sc_scatter(m4.reshape(E, EMB), dst2d, zeros_acc)
        hidden = _gru(aggp, hidden, r1(nn_bias), wih_t, r1(gru_bih), whh_t,
                      r1(gru_bhh))
        h = hidden
    return _decode(hidden, dec_W1, r1(dec_b1), r11(dec_a1), dec_W2, r1(dec_b2),
                   r11(dec_a2), dec_W3, r1(dec_b3), r11(dec_a3), dec_W4,
                   r1(dec_b4))
